# Initial kernel scaffold; baseline (speedup 1.0000x reference)
import jax
import jax.numpy as jnp
from jax.experimental import pallas as pl


def kernel(xyz, points, params):
    raise NotImplementedError("write your pallas kernel here")


# speedup vs baseline: 7.0953x; 7.0953x over previous
import functools

import jax
import jax.numpy as jnp
from jax import lax
from jax.experimental import pallas as pl
from jax.experimental.pallas import tpu as pltpu
from jax.experimental.pallas import tpu_sc as plsc

B = 8
N = 2048
NP = 512
NS = 32
TD = 80
ROWS = B * NP * NS
GROUPS = B * NP
EPS = 1e-5
BW = 0.1

_f32 = jnp.float32
_i32 = jnp.int32


def _density_body(xt_ref, x_ref, out_ref):
    a = xt_ref[0]
    bm = x_ref[0]
    g = jnp.dot(a, bm, preferred_element_type=_f32)
    srcn = jnp.sum(a * a, axis=-1, keepdims=True)
    dstn = jnp.sum(bm * bm, axis=0, keepdims=True)
    sq = -2.0 * g + srcn + dstn
    e = jnp.exp(sq * (-2.0 * BW * BW))
    dens = jnp.sum(e, axis=-1, keepdims=True) * (1.0 / (2.5 * BW) / N)
    out_ref[0] = 1.0 / dens


def _density(xyz_t, xyz):
    RB = 256
    return pl.pallas_call(
        _density_body,
        grid=(B, N // RB),
        in_specs=[
            pl.BlockSpec((1, RB, 3), lambda b, r: (b, r, 0)),
            pl.BlockSpec((1, 3, N), lambda b, r: (b, 0, 0)),
        ],
        out_specs=pl.BlockSpec((1, RB, 1), lambda b, r: (b, r, 0)),
        out_shape=jax.ShapeDtypeStruct((B, N, 1), _f32),
    )(xyz_t, xyz)


def _fps_body(x_ref, out_ref):
    x3 = x_ref[...]
    lane = lax.broadcasted_iota(_i32, (1, N), 1)
    lane_b = lax.broadcasted_iota(_i32, (B, N), 1)
    lane_p = lax.broadcasted_iota(_i32, (1, 1, NP), 2)
    out_ref[...] = jnp.zeros((3, B, NP), _f32)

    def body(i, st):
        dist, far = st
        oh = (lane_b == far).astype(_f32)
        c = jnp.sum(x3 * oh[None], axis=2, keepdims=True)
        d = jnp.sum((x3 - c) ** 2, axis=0)
        dist = jnp.minimum(dist, d)
        ohp = (lane_p == i).astype(_f32)
        out_ref[...] += c * ohp
        m = jnp.max(dist, axis=-1, keepdims=True)
        far = jnp.min(jnp.where(dist == m, lane_b, N), axis=-1, keepdims=True)
        return dist, far

    dist0 = jnp.full((B, N), 1e10, _f32)
    far0 = jnp.zeros((B, 1), _i32)
    lax.fori_loop(0, NP, body, (dist0, far0))


def _fps(xyz_bt):
    return pl.pallas_call(
        _fps_body,
        out_shape=jax.ShapeDtypeStruct((3, B, NP), _f32),
    )(xyz_bt)


def _knn_body(q_ref, x_ref, out_ref):
    b = pl.program_id(0)
    a = q_ref[0]
    bm = x_ref[0]
    g = jnp.dot(a, bm, preferred_element_type=_f32)
    srcn = jnp.sum(a * a, axis=-1, keepdims=True)
    dstn = jnp.sum(bm * bm, axis=0, keepdims=True)
    sq = -2.0 * g + srcn + dstn
    lane = lax.broadcasted_iota(_i32, sq.shape, 1)
    cols = []
    for _ in range(NS):
        m = jnp.min(sq, axis=-1, keepdims=True)
        sel = jnp.min(jnp.where(sq == m, lane, N), axis=-1, keepdims=True)
        cols.append(sel)
        sq = jnp.where(lane == sel, jnp.inf, sq)
    out_ref[0] = jnp.concatenate(cols, axis=1) + b * N


def _knn(new_xyz_q, xyz):
    QB = 128
    return pl.pallas_call(
        _knn_body,
        grid=(B, NP // QB),
        in_specs=[
            pl.BlockSpec((1, QB, 3), lambda b, q: (b, q, 0)),
            pl.BlockSpec((1, 3, N), lambda b, q: (b, 0, 0)),
        ],
        out_specs=pl.BlockSpec((1, QB, NS), lambda b, q: (b, q, 0)),
        out_shape=jax.ShapeDtypeStruct((B, NP, NS), _i32),
    )(new_xyz_q, xyz)


_NW = 32
_PERW = ROWS // _NW
_CH = 128


def _gather_body(table_hbm, idx_hbm, out_hbm, idx_v, rows_v, sem):
    wid = lax.axis_index("s") * 2 + lax.axis_index("c")

    def chunk(i, carry):
        base = wid * _PERW + i * _CH
        pltpu.sync_copy(idx_hbm.at[pl.ds(base, _CH)], idx_v)
        pltpu.async_copy(table_hbm.at[idx_v], rows_v, sem).wait()
        pltpu.sync_copy(rows_v, out_hbm.at[pl.ds(base, _CH)])
        return carry

    lax.fori_loop(0, _PERW // _CH, chunk, 0)


def _sc_gather(table, gidx):
    mesh = plsc.VectorSubcoreMesh(core_axis_name="c", subcore_axis_name="s")
    f = functools.partial(
        pl.kernel,
        mesh=mesh,
        out_type=jax.ShapeDtypeStruct((ROWS, TD), _f32),
        scratch_types=[
            pltpu.VMEM((_CH,), _i32),
            pltpu.VMEM((_CH, TD), _f32),
            pltpu.SemaphoreType.DMA,
        ],
        compiler_params=pltpu.CompilerParams(use_tc_tiling_on_sc=False),
    )(_gather_body)
    return f(table, gidx)


def _acc_stats(ref, y):
    @pl.when(pl.program_id(0) == 0)
    def _():
        ref[...] = jnp.zeros_like(ref)
    ref[0:1, :] += jnp.sum(y, axis=0, keepdims=True)
    ref[1:2, :] += jnp.sum(y * y, axis=0, keepdims=True)


def _bn_relu(y, st_ref, g_ref, be_ref, n):
    m = st_ref[0:1, :] * (1.0 / n)
    v = st_ref[1:2, :] * (1.0 / n) - m * m
    scale = g_ref[...] * lax.rsqrt(v + EPS)
    shift = be_ref[...] - m * scale
    return jnp.maximum(y * scale + shift, 0.0)


RB = 2048


def _p1_body(g_ref, nxe_ref, w1p_ref, w1a_ref, b1_ref, ww1_ref,
             y1_ref, wz1_ref, gd_ref, sy1_ref, sw1_ref):
    g = g_ref[...]
    nxe = nxe_ref[...]
    y1 = (jnp.dot(g, w1p_ref[...], preferred_element_type=_f32)
          - jnp.dot(nxe, w1a_ref[...], preferred_element_type=_f32)
          + b1_ref[...])
    gxn = g[:, 0:3] - nxe
    wz1 = jnp.dot(gxn, ww1_ref[...], preferred_element_type=_f32)
    y1_ref[...] = y1
    wz1_ref[...] = wz1
    gd_ref[...] = g[:, 67:68]
    _acc_stats(sy1_ref, y1)
    _acc_stats(sw1_ref, wz1)


def _p1(gathered, nxe, w1p, w1a, b1, ww1):
    grid = (ROWS // RB,)
    row = lambda r: (r, 0)
    fixed = lambda r: (0, 0)
    return pl.pallas_call(
        _p1_body,
        grid=grid,
        in_specs=[
            pl.BlockSpec((RB, TD), row),
            pl.BlockSpec((RB, 3), row),
            pl.BlockSpec(w1p.shape, fixed),
            pl.BlockSpec(w1a.shape, fixed),
            pl.BlockSpec(b1.shape, fixed),
            pl.BlockSpec(ww1.shape, fixed),
        ],
        out_specs=[
            pl.BlockSpec((RB, 64), row),
            pl.BlockSpec((RB, 8), row),
            pl.BlockSpec((RB, 1), row),
            pl.BlockSpec((8, 64), fixed),
            pl.BlockSpec((8, 8), fixed),
        ],
        out_shape=[
            jax.ShapeDtypeStruct((ROWS, 64), _f32),
            jax.ShapeDtypeStruct((ROWS, 8), _f32),
            jax.ShapeDtypeStruct((ROWS, 1), _f32),
            jax.ShapeDtypeStruct((8, 64), _f32),
            jax.ShapeDtypeStruct((8, 8), _f32),
        ],
    )(gathered, nxe, w1p, w1a, b1, ww1)


def _gmax_body(gd_ref, out_ref):
    gd = gd_ref[...]
    out_ref[...] = gd / jnp.max(gd, axis=-1, keepdims=True)


def _gmax(gd2):
    GB = 512
    return pl.pallas_call(
        _gmax_body,
        grid=(GROUPS // GB,),
        in_specs=[pl.BlockSpec((GB, NS), lambda r: (r, 0))],
        out_specs=pl.BlockSpec((GB, NS), lambda r: (r, 0)),
        out_shape=jax.ShapeDtypeStruct((GROUPS, NS), _f32),
    )(gd2)


def _p2_body(y1_ref, sy1_ref, g1_ref, be1_ref, w2_ref, b2_ref,
             wz1_ref, sw1_ref, wg1_ref, wbe1_ref, ww2_ref,
             gdn_ref, wd1_ref, db1_ref,
             y2_ref, wz2_ref, dz1_ref, sy2_ref, sw2_ref, sd1_ref):
    h1 = _bn_relu(y1_ref[...], sy1_ref, g1_ref, be1_ref, ROWS)
    y2 = jnp.dot(h1, w2_ref[...], preferred_element_type=_f32) + b2_ref[...]
    hw1 = _bn_relu(wz1_ref[...], sw1_ref, wg1_ref, wbe1_ref, ROWS)
    wz2 = jnp.dot(hw1, ww2_ref[...], preferred_element_type=_f32)
    dz1 = gdn_ref[...] * wd1_ref[...] + db1_ref[...]
    y2_ref[...] = y2
    wz2_ref[...] = wz2
    dz1_ref[...] = dz1
    _acc_stats(sy2_ref, y2)
    _acc_stats(sw2_ref, wz2)
    _acc_stats(sd1_ref, dz1)


def _p2(y1, sy1, g1, be1, w2, b2, wz1, sw1, wg1, wbe1, ww2, gdn, wd1, db1):
    row = lambda r: (r, 0)
    fixed = lambda r: (0, 0)
    args = (y1, sy1, g1, be1, w2, b2, wz1, sw1, wg1, wbe1, ww2, gdn, wd1, db1)
    blocks = [(RB, 64), None, None, None, None, None,
              (RB, 8), None, None, None, None, (RB, 1), None, None]
    in_specs = [
        pl.BlockSpec(bs if bs is not None else a.shape,
                     row if bs is not None else fixed)
        for a, bs in zip(args, blocks)
    ]
    return pl.pallas_call(
        _p2_body,
        grid=(ROWS // RB,),
        in_specs=in_specs,
        out_specs=[
            pl.BlockSpec((RB, 128), row),
            pl.BlockSpec((RB, 8), row),
            pl.BlockSpec((RB, 16), row),
            pl.BlockSpec((8, 128), fixed),
            pl.BlockSpec((8, 8), fixed),
            pl.BlockSpec((8, 16), fixed),
        ],
        out_shape=[
            jax.ShapeDtypeStruct((ROWS, 128), _f32),
            jax.ShapeDtypeStruct((ROWS, 8), _f32),
            jax.ShapeDtypeStruct((ROWS, 16), _f32),
            jax.ShapeDtypeStruct((8, 128), _f32),
            jax.ShapeDtypeStruct((8, 8), _f32),
            jax.ShapeDtypeStruct((8, 16), _f32),
        ],
    )(*args)


def _p3_body(wz2_ref, sw2_ref, wg2_ref, wbe2_ref, ww3_ref,
             dz1_ref, sd1_ref, dg1_ref, dbe1_ref, wd2_ref,
             wz3_ref, dz2_ref, sw3_ref, sd2_ref):
    hw2 = _bn_relu(wz2_ref[...], sw2_ref, wg2_ref, wbe2_ref, ROWS)
    wz3 = jnp.dot(hw2, ww3_ref[...], preferred_element_type=_f32)
    hd1 = _bn_relu(dz1_ref[...], sd1_ref, dg1_ref, dbe1_ref, ROWS)
    dz2 = jnp.dot(hd1, wd2_ref[...], preferred_element_type=_f32)
    wz3_ref[...] = wz3
    dz2_ref[...] = dz2
    _acc_stats(sw3_ref, wz3)
    _acc_stats(sd2_ref, dz2)


def _p3(wz2, sw2, wg2, wbe2, ww3, dz1, sd1, dg1, dbe1, wd2):
    row = lambda r: (r, 0)
    fixed = lambda r: (0, 0)
    args = (wz2, sw2, wg2, wbe2, ww3, dz1, sd1, dg1, dbe1, wd2)
    blocks = [(RB, 8), None, None, None, None, (RB, 16), None, None, None, None]
    in_specs = [
        pl.BlockSpec(bs if bs is not None else a.shape,
                     row if bs is not None else fixed)
        for a, bs in zip(args, blocks)
    ]
    return pl.pallas_call(
        _p3_body,
        grid=(ROWS // RB,),
        in_specs=in_specs,
        out_specs=[
            pl.BlockSpec((RB, 16), row),
            pl.BlockSpec((RB, 8), row),
            pl.BlockSpec((8, 16), fixed),
            pl.BlockSpec((8, 8), fixed),
        ],
        out_shape=[
            jax.ShapeDtypeStruct((ROWS, 16), _f32),
            jax.ShapeDtypeStruct((ROWS, 8), _f32),
            jax.ShapeDtypeStruct((8, 16), _f32),
            jax.ShapeDtypeStruct((8, 8), _f32),
        ],
    )(*args)


def _p4_body(dz2_ref, sd2_ref, dg2_ref, dbe2_ref, wd3_ref,
             dz3_ref, sd3_ref):
    hd2 = _bn_relu(dz2_ref[...], sd2_ref, dg2_ref, dbe2_ref, ROWS)
    dz3 = jnp.dot(hd2, wd3_ref[...], preferred_element_type=_f32)
    dz3_ref[...] = dz3
    _acc_stats(sd3_ref, dz3)


def _p4(dz2, sd2, dg2, dbe2, wd3):
    row = lambda r: (r, 0)
    fixed = lambda r: (0, 0)
    args = (dz2, sd2, dg2, dbe2, wd3)
    blocks = [(RB, 8), None, None, None, None]
    in_specs = [
        pl.BlockSpec(bs if bs is not None else a.shape,
                     row if bs is not None else fixed)
        for a, bs in zip(args, blocks)
    ]
    return pl.pallas_call(
        _p4_body,
        grid=(ROWS // RB,),
        in_specs=in_specs,
        out_specs=[
            pl.BlockSpec((RB, 1), row),
            pl.BlockSpec((8, 1), fixed),
        ],
        out_shape=[
            jax.ShapeDtypeStruct((ROWS, 1), _f32),
            jax.ShapeDtypeStruct((8, 1), _f32),
        ],
    )(*args)


SB = 128
RB5 = SB * NS


def _p5_body(y2_ref, sy2_ref, g2_ref, be2_ref,
             dz3_ref, sd3_ref, dg3_ref, dbe3_ref,
             wz3_ref, sw3_ref, wg3_ref, wbe3_ref,
             lstack_ref, lb_ref,
             out_ref, so_ref):
    h2 = _bn_relu(y2_ref[...], sy2_ref, g2_ref, be2_ref, ROWS)
    ds = _bn_relu(dz3_ref[...], sd3_ref, dg3_ref, dbe3_ref, ROWS)
    wv = _bn_relu(wz3_ref[...], sw3_ref, wg3_ref, wbe3_ref, ROWS)
    x = h2 * ds
    xr = x.reshape(SB, NS, 128)
    wr = wv.reshape(SB, NS, 16)
    acc = jnp.zeros((SB, 128), _f32) + lb_ref[...]
    for j in range(16):
        wj = wr[:, :, j:j + 1]
        mj = jnp.sum(xr * wj, axis=1)
        acc += jnp.dot(mj, lstack_ref[j], preferred_element_type=_f32)
    out_ref[...] = acc
    _acc_stats(so_ref, acc)


def _p5(y2, sy2, g2, be2, dz3, sd3, dg3, dbe3, wz3, sw3, wg3, wbe3,
        lstack, lb):
    fixed = lambda r: (0, 0)
    args = (y2, sy2, g2, be2, dz3, sd3, dg3, dbe3, wz3, sw3, wg3, wbe3, lb)
    blocks = [(RB5, 128), None, None, None, (RB5, 1), None, None, None,
              (RB5, 16), None, None, None, None]
    in_specs = [
        pl.BlockSpec(bs if bs is not None else a.shape,
                     (lambda r: (r, 0)) if bs is not None else fixed)
        for a, bs in zip(args, blocks)
    ]
    in_specs = in_specs[:-1] + [
        pl.BlockSpec(lstack.shape, lambda r: (0, 0, 0)),
        in_specs[-1],
    ]
    return pl.pallas_call(
        _p5_body,
        grid=(ROWS // RB5,),
        in_specs=in_specs,
        out_specs=[
            pl.BlockSpec((SB, 128), lambda r: (r, 0)),
            pl.BlockSpec((8, 128), fixed),
        ],
        out_shape=[
            jax.ShapeDtypeStruct((GROUPS, 128), _f32),
            jax.ShapeDtypeStruct((8, 128), _f32),
        ],
    )(y2, sy2, g2, be2, dz3, sd3, dg3, dbe3, wz3, sw3, wg3, wbe3, lstack, lb)


def _p6_body(o_ref, so_ref, g_ref, be_ref, out_ref):
    out_ref[...] = _bn_relu(o_ref[...], so_ref, g_ref, be_ref, GROUPS)


def _p6(out_pre, so, g, be):
    OB = 512
    fixed = lambda r: (0, 0)
    return pl.pallas_call(
        _p6_body,
        grid=(GROUPS // OB,),
        in_specs=[
            pl.BlockSpec((OB, 128), lambda r: (r, 0)),
            pl.BlockSpec(so.shape, fixed),
            pl.BlockSpec(g.shape, fixed),
            pl.BlockSpec(be.shape, fixed),
        ],
        out_specs=pl.BlockSpec((OB, 128), lambda r: (r, 0)),
        out_shape=jax.ShapeDtypeStruct((GROUPS, 128), _f32),
    )(out_pre, so, g, be)


def kernel(xyz, points, params):
    xyz_t = jnp.transpose(xyz, (0, 2, 1))
    pts_t = jnp.transpose(points, (0, 2, 1))

    inv_d = _density(xyz_t, xyz)

    new_xyz_sc = _fps(jnp.transpose(xyz, (1, 0, 2)))
    new_xyz = jnp.transpose(new_xyz_sc, (1, 2, 0))

    idx = _knn(new_xyz, xyz)

    table = jnp.concatenate(
        [xyz_t, pts_t, inv_d, jnp.zeros((B, N, TD - 68), _f32)], axis=-1,
    ).reshape(B * N, TD)
    gathered = _sc_gather(table, idx.reshape(ROWS))

    nxe = jnp.broadcast_to(new_xyz[:, :, None, :],
                           (B, NP, NS, 3)).reshape(ROWS, 3)

    p = params
    mlp0, mlp1 = p['mlp'][0], p['mlp'][1]
    wn0, wn1, wn2 = p['wn']
    dn0, dn1, dn2 = p['dn']

    w1p = jnp.concatenate(
        [mlp0['w'], jnp.zeros((64, TD - 67), _f32)], axis=1).T
    w1a = mlp0['w'][:, 0:3].T
    r1 = lambda a: a.reshape(1, -1)

    y1, wz1, gd, sy1, sw1 = _p1(gathered, nxe, w1p, w1a,
                                r1(mlp0['b']), wn0['w'].T)

    gdn = _gmax(gd.reshape(GROUPS, NS)).reshape(ROWS, 1)

    y2, wz2, dz1, sy2, sw2, sd1 = _p2(
        y1, sy1, r1(mlp0['g']), r1(mlp0['be']), mlp1['w'].T, r1(mlp1['b']),
        wz1, sw1, r1(wn0['g']), r1(wn0['be']), wn1['w'].T,
        gdn, dn0['w'].T, r1(dn0['b']))

    wz3, dz2, sw3, sd2 = _p3(
        wz2, sw2, r1(wn1['g']), r1(wn1['be']), wn2['w'].T,
        dz1, sd1, r1(dn0['g']), r1(dn0['be']), dn1['w'].T)

    dz3, sd3 = _p4(dz2, sd2, r1(dn1['g']), r1(dn1['be']), dn2['w'].T)

    lstack = jnp.transpose(p['lin_w'].reshape(128, 128, 16), (2, 1, 0))
    out_pre, so = _p5(
        y2, sy2, r1(mlp1['g']), r1(mlp1['be']),
        dz3, sd3, r1(dn2['g']), r1(dn2['be']),
        wz3, sw3, r1(wn2['g']), r1(wn2['be']),
        lstack, r1(p['lin_b']))

    out = _p6(out_pre, so, r1(p['bnl_g']), r1(p['bnl_be']))
    out = jnp.transpose(out.reshape(B, NP, 128), (0, 2, 1))
    return jnp.transpose(new_xyz_sc, (1, 0, 2)), out


# speedup vs baseline: 8.1629x; 1.1505x over previous
import functools

import jax
import jax.numpy as jnp
from jax import lax
from jax.experimental import pallas as pl
from jax.experimental.pallas import tpu as pltpu
from jax.experimental.pallas import tpu_sc as plsc

B = 8
N = 2048
NP = 512
NS = 32
TD = 80
ROWS = B * NP * NS
GROUPS = B * NP
EPS = 1e-5
BW = 0.1

_f32 = jnp.float32
_i32 = jnp.int32


def _density_body(xt_ref, x_ref, out_ref):
    a = xt_ref[0]
    bm = x_ref[0]
    g = jnp.dot(a, bm, preferred_element_type=_f32)
    srcn = jnp.sum(a * a, axis=-1, keepdims=True)
    dstn = jnp.sum(bm * bm, axis=0, keepdims=True)
    sq = -2.0 * g + srcn + dstn
    e = jnp.exp(sq * (-2.0 * BW * BW))
    dens = jnp.sum(e, axis=-1, keepdims=True) * (1.0 / (2.5 * BW) / N)
    out_ref[0] = 1.0 / dens


def _density(xyz_t, xyz):
    RB = 256
    return pl.pallas_call(
        _density_body,
        grid=(B, N // RB),
        in_specs=[
            pl.BlockSpec((1, RB, 3), lambda b, r: (b, r, 0)),
            pl.BlockSpec((1, 3, N), lambda b, r: (b, 0, 0)),
        ],
        out_specs=pl.BlockSpec((1, RB, 1), lambda b, r: (b, r, 0)),
        out_shape=jax.ShapeDtypeStruct((B, N, 1), _f32),
    )(xyz_t, xyz)


def _fps_body(x_ref, out_ref):
    x3 = x_ref[...]
    lane = lax.broadcasted_iota(_i32, (1, N), 1)
    lane_b = lax.broadcasted_iota(_i32, (B, N), 1)
    lane_p = lax.broadcasted_iota(_i32, (1, 1, NP), 2)
    out_ref[...] = jnp.zeros((3, B, NP), _f32)

    def body(i, st):
        dist, far = st
        oh = (lane_b == far).astype(_f32)
        c = jnp.sum(x3 * oh[None], axis=2, keepdims=True)
        d = jnp.sum((x3 - c) ** 2, axis=0)
        dist = jnp.minimum(dist, d)
        ohp = (lane_p == i).astype(_f32)
        out_ref[...] += c * ohp
        m = jnp.max(dist, axis=-1, keepdims=True)
        far = jnp.min(jnp.where(dist == m, lane_b, N), axis=-1, keepdims=True)
        return dist, far

    dist0 = jnp.full((B, N), 1e10, _f32)
    far0 = jnp.zeros((B, 1), _i32)
    lax.fori_loop(0, NP, body, (dist0, far0))


def _fps(xyz_bt):
    return pl.pallas_call(
        _fps_body,
        out_shape=jax.ShapeDtypeStruct((3, B, NP), _f32),
    )(xyz_bt)


def _knn_body(q_ref, x_ref, out_ref):
    b = pl.program_id(0)
    a = q_ref[0]
    bm = x_ref[0]
    g = jnp.dot(a, bm, preferred_element_type=_f32)
    srcn = jnp.sum(a * a, axis=-1, keepdims=True)
    dstn = jnp.sum(bm * bm, axis=0, keepdims=True)
    sq = -2.0 * g + srcn + dstn
    lane = lax.broadcasted_iota(_i32, sq.shape, 1)
    cols = []
    for _ in range(NS):
        m = jnp.min(sq, axis=-1, keepdims=True)
        sel = jnp.min(jnp.where(sq == m, lane, N), axis=-1, keepdims=True)
        cols.append(sel)
        sq = jnp.where(lane == sel, jnp.inf, sq)
    out_ref[0] = jnp.concatenate(cols, axis=1) + b * N


def _knn(new_xyz_q, xyz):
    QB = 128
    return pl.pallas_call(
        _knn_body,
        grid=(B, NP // QB),
        in_specs=[
            pl.BlockSpec((1, QB, 3), lambda b, q: (b, q, 0)),
            pl.BlockSpec((1, 3, N), lambda b, q: (b, 0, 0)),
        ],
        out_specs=pl.BlockSpec((1, QB, NS), lambda b, q: (b, q, 0)),
        out_shape=jax.ShapeDtypeStruct((B, NP, NS), _i32),
    )(new_xyz_q, xyz)


_NW = 32
_PERW = ROWS // _NW
_CH = 128


def _gather_body(table_hbm, idx_hbm, out_hbm, idx_v, rows_v, sem):
    wid = lax.axis_index("s") * 2 + lax.axis_index("c")

    def chunk(i, carry):
        base = wid * _PERW + i * _CH
        pltpu.sync_copy(idx_hbm.at[pl.ds(base, _CH)], idx_v)
        pltpu.async_copy(table_hbm.at[idx_v], rows_v, sem).wait()
        pltpu.sync_copy(rows_v, out_hbm.at[pl.ds(base, _CH)])
        return carry

    lax.fori_loop(0, _PERW // _CH, chunk, 0)


def _sc_gather(table, gidx):
    mesh = plsc.VectorSubcoreMesh(core_axis_name="c", subcore_axis_name="s")
    f = functools.partial(
        pl.kernel,
        mesh=mesh,
        out_type=jax.ShapeDtypeStruct((ROWS, TD), _f32),
        scratch_types=[
            pltpu.VMEM((_CH,), _i32),
            pltpu.VMEM((_CH, TD), _f32),
            pltpu.SemaphoreType.DMA,
        ],
        compiler_params=pltpu.CompilerParams(use_tc_tiling_on_sc=False),
    )(_gather_body)
    return f(table, gidx)


def _acc_stats(ref, y):
    @pl.when(pl.program_id(0) == 0)
    def _():
        ref[...] = jnp.zeros_like(ref)
    ones = jnp.ones((1, y.shape[0]), _f32)
    ref[0:1, :] += jnp.dot(ones, y, preferred_element_type=_f32)
    ref[1:2, :] += jnp.dot(ones, y * y, preferred_element_type=_f32)


def _bn_relu(y, st_ref, g_ref, be_ref, n):
    m = st_ref[0:1, :] * (1.0 / n)
    v = st_ref[1:2, :] * (1.0 / n) - m * m
    scale = g_ref[...] * lax.rsqrt(v + EPS)
    shift = be_ref[...] - m * scale
    return jnp.maximum(y * scale + shift, 0.0)


RB = 2048


def _p1_body(g_ref, nxe_ref, w1p_ref, w1a_ref, b1_ref, ww1_ref,
             y1_ref, wz1_ref, gd_ref, sy1_ref, sw1_ref):
    g = g_ref[...]
    nxe = nxe_ref[...]
    y1 = (jnp.dot(g, w1p_ref[...], preferred_element_type=_f32)
          - jnp.dot(nxe, w1a_ref[...], preferred_element_type=_f32)
          + b1_ref[...])
    gxn = g[:, 0:3] - nxe
    wz1 = jnp.dot(gxn, ww1_ref[...], preferred_element_type=_f32)
    y1_ref[...] = y1
    wz1_ref[...] = wz1
    gd_ref[...] = g[:, 67:68]
    _acc_stats(sy1_ref, y1)
    _acc_stats(sw1_ref, wz1)


def _p1(gathered, nxe, w1p, w1a, b1, ww1):
    grid = (ROWS // RB,)
    row = lambda r: (r, 0)
    fixed = lambda r: (0, 0)
    return pl.pallas_call(
        _p1_body,
        grid=grid,
        in_specs=[
            pl.BlockSpec((RB, TD), row),
            pl.BlockSpec((RB, 3), row),
            pl.BlockSpec(w1p.shape, fixed),
            pl.BlockSpec(w1a.shape, fixed),
            pl.BlockSpec(b1.shape, fixed),
            pl.BlockSpec(ww1.shape, fixed),
        ],
        out_specs=[
            pl.BlockSpec((RB, 64), row),
            pl.BlockSpec((RB, 8), row),
            pl.BlockSpec((RB, 1), row),
            pl.BlockSpec((8, 64), fixed),
            pl.BlockSpec((8, 8), fixed),
        ],
        out_shape=[
            jax.ShapeDtypeStruct((ROWS, 64), _f32),
            jax.ShapeDtypeStruct((ROWS, 8), _f32),
            jax.ShapeDtypeStruct((ROWS, 1), _f32),
            jax.ShapeDtypeStruct((8, 64), _f32),
            jax.ShapeDtypeStruct((8, 8), _f32),
        ],
    )(gathered, nxe, w1p, w1a, b1, ww1)


def _gmax_body(gd_ref, out_ref):
    gd = gd_ref[...]
    out_ref[...] = gd / jnp.max(gd, axis=-1, keepdims=True)


def _gmax(gd2):
    GB = 512
    return pl.pallas_call(
        _gmax_body,
        grid=(GROUPS // GB,),
        in_specs=[pl.BlockSpec((GB, NS), lambda r: (r, 0))],
        out_specs=pl.BlockSpec((GB, NS), lambda r: (r, 0)),
        out_shape=jax.ShapeDtypeStruct((GROUPS, NS), _f32),
    )(gd2)


def _p2_body(y1_ref, sy1_ref, g1_ref, be1_ref, w2_ref, b2_ref,
             wz1_ref, sw1_ref, wg1_ref, wbe1_ref, ww2_ref,
             gdn_ref, wd1_ref, db1_ref,
             y2_ref, wz2_ref, dz1_ref, sy2_ref, sw2_ref, sd1_ref):
    h1 = _bn_relu(y1_ref[...], sy1_ref, g1_ref, be1_ref, ROWS)
    y2 = jnp.dot(h1, w2_ref[...], preferred_element_type=_f32) + b2_ref[...]
    hw1 = _bn_relu(wz1_ref[...], sw1_ref, wg1_ref, wbe1_ref, ROWS)
    wz2 = jnp.dot(hw1, ww2_ref[...], preferred_element_type=_f32)
    dz1 = gdn_ref[...] * wd1_ref[...] + db1_ref[...]
    y2_ref[...] = y2
    wz2_ref[...] = wz2
    dz1_ref[...] = dz1
    _acc_stats(sy2_ref, y2)
    _acc_stats(sw2_ref, wz2)
    _acc_stats(sd1_ref, dz1)


def _p2(y1, sy1, g1, be1, w2, b2, wz1, sw1, wg1, wbe1, ww2, gdn, wd1, db1):
    row = lambda r: (r, 0)
    fixed = lambda r: (0, 0)
    args = (y1, sy1, g1, be1, w2, b2, wz1, sw1, wg1, wbe1, ww2, gdn, wd1, db1)
    blocks = [(RB, 64), None, None, None, None, None,
              (RB, 8), None, None, None, None, (RB, 1), None, None]
    in_specs = [
        pl.BlockSpec(bs if bs is not None else a.shape,
                     row if bs is not None else fixed)
        for a, bs in zip(args, blocks)
    ]
    return pl.pallas_call(
        _p2_body,
        grid=(ROWS // RB,),
        in_specs=in_specs,
        out_specs=[
            pl.BlockSpec((RB, 128), row),
            pl.BlockSpec((RB, 8), row),
            pl.BlockSpec((RB, 16), row),
            pl.BlockSpec((8, 128), fixed),
            pl.BlockSpec((8, 8), fixed),
            pl.BlockSpec((8, 16), fixed),
        ],
        out_shape=[
            jax.ShapeDtypeStruct((ROWS, 128), _f32),
            jax.ShapeDtypeStruct((ROWS, 8), _f32),
            jax.ShapeDtypeStruct((ROWS, 16), _f32),
            jax.ShapeDtypeStruct((8, 128), _f32),
            jax.ShapeDtypeStruct((8, 8), _f32),
            jax.ShapeDtypeStruct((8, 16), _f32),
        ],
    )(*args)


def _p3_body(wz2_ref, sw2_ref, wg2_ref, wbe2_ref, ww3_ref,
             dz1_ref, sd1_ref, dg1_ref, dbe1_ref, wd2_ref,
             wz3_ref, dz2_ref, sw3_ref, sd2_ref):
    hw2 = _bn_relu(wz2_ref[...], sw2_ref, wg2_ref, wbe2_ref, ROWS)
    wz3 = jnp.dot(hw2, ww3_ref[...], preferred_element_type=_f32)
    hd1 = _bn_relu(dz1_ref[...], sd1_ref, dg1_ref, dbe1_ref, ROWS)
    dz2 = jnp.dot(hd1, wd2_ref[...], preferred_element_type=_f32)
    wz3_ref[...] = wz3
    dz2_ref[...] = dz2
    _acc_stats(sw3_ref, wz3)
    _acc_stats(sd2_ref, dz2)


def _p3(wz2, sw2, wg2, wbe2, ww3, dz1, sd1, dg1, dbe1, wd2):
    row = lambda r: (r, 0)
    fixed = lambda r: (0, 0)
    args = (wz2, sw2, wg2, wbe2, ww3, dz1, sd1, dg1, dbe1, wd2)
    blocks = [(RB, 8), None, None, None, None, (RB, 16), None, None, None, None]
    in_specs = [
        pl.BlockSpec(bs if bs is not None else a.shape,
                     row if bs is not None else fixed)
        for a, bs in zip(args, blocks)
    ]
    return pl.pallas_call(
        _p3_body,
        grid=(ROWS // RB,),
        in_specs=in_specs,
        out_specs=[
            pl.BlockSpec((RB, 16), row),
            pl.BlockSpec((RB, 8), row),
            pl.BlockSpec((8, 16), fixed),
            pl.BlockSpec((8, 8), fixed),
        ],
        out_shape=[
            jax.ShapeDtypeStruct((ROWS, 16), _f32),
            jax.ShapeDtypeStruct((ROWS, 8), _f32),
            jax.ShapeDtypeStruct((8, 16), _f32),
            jax.ShapeDtypeStruct((8, 8), _f32),
        ],
    )(*args)


def _p4_body(dz2_ref, sd2_ref, dg2_ref, dbe2_ref, wd3_ref,
             dz3_ref, sd3_ref):
    hd2 = _bn_relu(dz2_ref[...], sd2_ref, dg2_ref, dbe2_ref, ROWS)
    dz3 = jnp.dot(hd2, wd3_ref[...], preferred_element_type=_f32)
    dz3_ref[...] = dz3
    _acc_stats(sd3_ref, dz3)


def _p4(dz2, sd2, dg2, dbe2, wd3):
    row = lambda r: (r, 0)
    fixed = lambda r: (0, 0)
    args = (dz2, sd2, dg2, dbe2, wd3)
    blocks = [(RB, 8), None, None, None, None]
    in_specs = [
        pl.BlockSpec(bs if bs is not None else a.shape,
                     row if bs is not None else fixed)
        for a, bs in zip(args, blocks)
    ]
    return pl.pallas_call(
        _p4_body,
        grid=(ROWS // RB,),
        in_specs=in_specs,
        out_specs=[
            pl.BlockSpec((RB, 1), row),
            pl.BlockSpec((8, 1), fixed),
        ],
        out_shape=[
            jax.ShapeDtypeStruct((ROWS, 1), _f32),
            jax.ShapeDtypeStruct((8, 1), _f32),
        ],
    )(*args)


SB = 128
RB5 = SB * NS


def _p5_body(y2_ref, sy2_ref, g2_ref, be2_ref,
             dz3_ref, sd3_ref, dg3_ref, dbe3_ref,
             wz3_ref, sw3_ref, wg3_ref, wbe3_ref,
             lstack_ref, lb_ref,
             out_ref, so_ref):
    h2 = _bn_relu(y2_ref[...], sy2_ref, g2_ref, be2_ref, ROWS)
    ds = _bn_relu(dz3_ref[...], sd3_ref, dg3_ref, dbe3_ref, ROWS)
    wv = _bn_relu(wz3_ref[...], sw3_ref, wg3_ref, wbe3_ref, ROWS)
    x = h2 * ds
    xr = x.reshape(SB, NS, 128)
    wr = wv.reshape(SB, NS, 16)
    m = lax.dot_general(wr, xr,
                        dimension_numbers=(((1,), (1,)), ((0,), (0,))),
                        preferred_element_type=_f32)
    acc = jnp.zeros((SB, 128), _f32) + lb_ref[...]
    for j in range(16):
        acc += jnp.dot(m[:, j, :], lstack_ref[j], preferred_element_type=_f32)
    out_ref[...] = acc
    _acc_stats(so_ref, acc)


def _p5(y2, sy2, g2, be2, dz3, sd3, dg3, dbe3, wz3, sw3, wg3, wbe3,
        lstack, lb):
    fixed = lambda r: (0, 0)
    args = (y2, sy2, g2, be2, dz3, sd3, dg3, dbe3, wz3, sw3, wg3, wbe3, lb)
    blocks = [(RB5, 128), None, None, None, (RB5, 1), None, None, None,
              (RB5, 16), None, None, None, None]
    in_specs = [
        pl.BlockSpec(bs if bs is not None else a.shape,
                     (lambda r: (r, 0)) if bs is not None else fixed)
        for a, bs in zip(args, blocks)
    ]
    in_specs = in_specs[:-1] + [
        pl.BlockSpec(lstack.shape, lambda r: (0, 0, 0)),
        in_specs[-1],
    ]
    return pl.pallas_call(
        _p5_body,
        grid=(ROWS // RB5,),
        in_specs=in_specs,
        out_specs=[
            pl.BlockSpec((SB, 128), lambda r: (r, 0)),
            pl.BlockSpec((8, 128), fixed),
        ],
        out_shape=[
            jax.ShapeDtypeStruct((GROUPS, 128), _f32),
            jax.ShapeDtypeStruct((8, 128), _f32),
        ],
    )(y2, sy2, g2, be2, dz3, sd3, dg3, dbe3, wz3, sw3, wg3, wbe3, lstack, lb)


def _p6_body(o_ref, so_ref, g_ref, be_ref, out_ref):
    out_ref[...] = _bn_relu(o_ref[...], so_ref, g_ref, be_ref, GROUPS)


def _p6(out_pre, so, g, be):
    OB = 512
    fixed = lambda r: (0, 0)
    return pl.pallas_call(
        _p6_body,
        grid=(GROUPS // OB,),
        in_specs=[
            pl.BlockSpec((OB, 128), lambda r: (r, 0)),
            pl.BlockSpec(so.shape, fixed),
            pl.BlockSpec(g.shape, fixed),
            pl.BlockSpec(be.shape, fixed),
        ],
        out_specs=pl.BlockSpec((OB, 128), lambda r: (r, 0)),
        out_shape=jax.ShapeDtypeStruct((GROUPS, 128), _f32),
    )(out_pre, so, g, be)


def kernel(xyz, points, params):
    xyz_t = jnp.transpose(xyz, (0, 2, 1))
    pts_t = jnp.transpose(points, (0, 2, 1))

    inv_d = _density(xyz_t, xyz)

    new_xyz_sc = _fps(jnp.transpose(xyz, (1, 0, 2)))
    new_xyz = jnp.transpose(new_xyz_sc, (1, 2, 0))

    idx = _knn(new_xyz, xyz)

    table = jnp.concatenate(
        [xyz_t, pts_t, inv_d, jnp.zeros((B, N, TD - 68), _f32)], axis=-1,
    ).reshape(B * N, TD)
    gathered = _sc_gather(table, idx.reshape(ROWS))

    nxe = jnp.broadcast_to(new_xyz[:, :, None, :],
                           (B, NP, NS, 3)).reshape(ROWS, 3)

    p = params
    mlp0, mlp1 = p['mlp'][0], p['mlp'][1]
    wn0, wn1, wn2 = p['wn']
    dn0, dn1, dn2 = p['dn']

    w1p = jnp.concatenate(
        [mlp0['w'], jnp.zeros((64, TD - 67), _f32)], axis=1).T
    w1a = mlp0['w'][:, 0:3].T
    r1 = lambda a: a.reshape(1, -1)

    y1, wz1, gd, sy1, sw1 = _p1(gathered, nxe, w1p, w1a,
                                r1(mlp0['b']), wn0['w'].T)

    gdn = _gmax(gd.reshape(GROUPS, NS)).reshape(ROWS, 1)

    y2, wz2, dz1, sy2, sw2, sd1 = _p2(
        y1, sy1, r1(mlp0['g']), r1(mlp0['be']), mlp1['w'].T, r1(mlp1['b']),
        wz1, sw1, r1(wn0['g']), r1(wn0['be']), wn1['w'].T,
        gdn, dn0['w'].T, r1(dn0['b']))

    wz3, dz2, sw3, sd2 = _p3(
        wz2, sw2, r1(wn1['g']), r1(wn1['be']), wn2['w'].T,
        dz1, sd1, r1(dn0['g']), r1(dn0['be']), dn1['w'].T)

    dz3, sd3 = _p4(dz2, sd2, r1(dn1['g']), r1(dn1['be']), dn2['w'].T)

    lstack = jnp.transpose(p['lin_w'].reshape(128, 128, 16), (2, 1, 0))
    out_pre, so = _p5(
        y2, sy2, r1(mlp1['g']), r1(mlp1['be']),
        dz3, sd3, r1(dn2['g']), r1(dn2['be']),
        wz3, sw3, r1(wn2['g']), r1(wn2['be']),
        lstack, r1(p['lin_b']))

    out = _p6(out_pre, so, r1(p['bnl_g']), r1(p['bnl_be']))
    out = jnp.transpose(out.reshape(B, NP, 128), (0, 2, 1))
    return jnp.transpose(new_xyz_sc, (1, 0, 2)), out
